# baseline (speedup 1.0000x reference)
import jax
import jax.numpy as jnp
from jax.experimental import pallas as pl
from jax.experimental.pallas import tpu as pltpu

_H = 32
_BN = 4096


def _body(scalar_ref, hist_ref, act_ref, out_ref):
    shift = scalar_ref[0]
    pos = scalar_ref[1]
    x = hist_ref[0]
    rolled = pltpu.roll(x, shift, 1)
    a = act_ref[0, 0, 0]
    lane = jax.lax.broadcasted_iota(jnp.int32, x.shape, 1)
    out_ref[0] = jnp.where(lane == pos, a[:, None], rolled)


def kernel(history, activations, current_idx):
    B, N, H = history.shape
    idx = jnp.asarray(current_idx, dtype=jnp.int32)
    new_idx = idx + 1
    s = new_idx % H
    wrapped = new_idx >= H
    shift = jnp.where(wrapped, (H - s) % H, 0).astype(jnp.int32)
    pos = jnp.where(wrapped, H - 1, idx % H).astype(jnp.int32)
    scalars = jnp.stack([shift, pos])

    bn = _BN
    act3 = activations.reshape(B, N // bn, 1, bn)

    grid = (B, N // bn)
    out = pl.pallas_call(
        _body,
        grid_spec=pltpu.PrefetchScalarGridSpec(
            num_scalar_prefetch=1,
            grid=grid,
            in_specs=[
                pl.BlockSpec((1, bn, H), lambda b, n, *_: (b, n, 0)),
                pl.BlockSpec((1, 1, 1, bn), lambda b, n, *_: (b, n, 0, 0)),
            ],
            out_specs=pl.BlockSpec((1, bn, H), lambda b, n, *_: (b, n, 0)),
        ),
        out_shape=jax.ShapeDtypeStruct((B, N, H), history.dtype),
    )(scalars, history, act3)
    return out
